# baseline (speedup 1.0000x reference)
import functools

import jax
import jax.numpy as jnp
from jax import lax
from jax.experimental import pallas as pl
from jax.experimental.pallas import tpu as pltpu
from jax.experimental.pallas import tpu_sc as plsc

VOCAB = 1000000
D = 64
NCLS = 16
B = 16384
SEQ = 20
HID = 32

NC = 2
NS = 16
NW = NC * NS

BPW = B // NW
CHUNK_B = 32
NCHUNK = BPW // CHUNK_B
CHUNK_IDX = CHUNK_B * SEQ
GATHER_N = 128
NGATH = CHUNK_IDX // GATHER_N
IDX_ROWS = BPW * SEQ // GATHER_N


def _sc_body(x_hbm, table_hbm, out_hbm, idx_v, rows_v, out_v, sem0, sem1):
    cid = lax.axis_index("c")
    sid = lax.axis_index("s")
    wid = sid * NC + cid

    pltpu.sync_copy(x_hbm.at[wid], idx_v)

    def issue(c, par, sem):
        def g_body(g, carry):
            pltpu.async_copy(
                table_hbm.at[idx_v.at[c * NGATH + g]],
                rows_v.at[par, pl.ds(g * GATHER_N, GATHER_N)],
                sem,
            )
            return carry
        lax.fori_loop(0, NGATH, g_body, 0)

    def drain(par, sem):
        def w_body(g, carry):
            pltpu.make_async_copy(
                table_hbm.at[pl.ds(0, GATHER_N), :],
                rows_v.at[par, pl.ds(0, GATHER_N)],
                sem,
            ).wait()
            return carry
        lax.fori_loop(0, NGATH, w_body, 0)

    def pool(c, par):
        def b_body(b, carry):
            rb = b * SEQ
            for dd in range(D // 16):
                acc = rows_v[par, rb, pl.ds(dd * 16, 16)]
                for j in range(1, SEQ):
                    acc = acc + rows_v[par, rb + j, pl.ds(dd * 16, 16)]
                out_v[c * CHUNK_B + b, pl.ds(dd * 16, 16)] = acc
            return carry
        lax.fori_loop(0, CHUNK_B, b_body, 0)

    issue(0, 0, sem0)

    def outer(k, carry):
        c0 = 2 * k
        c1 = 2 * k + 1
        issue(c1, 1, sem1)
        drain(0, sem0)
        pool(c0, 0)

        @pl.when(c1 + 1 < NCHUNK)
        def _():
            issue(c1 + 1, 0, sem0)

        drain(1, sem1)
        pool(c1, 1)
        return carry

    lax.fori_loop(0, NCHUNK // 2, outer, 0)

    pltpu.sync_copy(out_v, out_hbm.at[pl.ds(wid * BPW, BPW), :])


_sc_pool = functools.partial(
    pl.kernel,
    out_type=jax.ShapeDtypeStruct((B, D), jnp.float32),
    mesh=plsc.VectorSubcoreMesh(core_axis_name="c", subcore_axis_name="s"),
    compiler_params=pltpu.CompilerParams(use_tc_tiling_on_sc=False),
    scratch_types=[
        pltpu.VMEM((IDX_ROWS, GATHER_N), jnp.int32),
        pltpu.VMEM((2, CHUNK_IDX, D), jnp.float32),
        pltpu.VMEM((BPW, D), jnp.float32),
        pltpu.SemaphoreType.DMA,
        pltpu.SemaphoreType.DMA,
    ],
)(_sc_body)


MB = 2048


def _mlp_body(p_ref, w1_ref, b1_ref, w2_ref, b2_ref, o_ref):
    p = p_ref[...] * (1.0 / SEQ)
    h = lax.dot_general(p, w1_ref[...], (((1,), (1,)), ((), ())),
                        preferred_element_type=jnp.float32)
    h = h + b1_ref[...]
    o = lax.dot_general(h, w2_ref[...], (((1,), (1,)), ((), ())),
                        preferred_element_type=jnp.float32)
    o_ref[...] = o + b2_ref[...]


_mlp = pl.pallas_call(
    _mlp_body,
    grid=(B // MB,),
    in_specs=[
        pl.BlockSpec((MB, D), lambda i: (i, 0)),
        pl.BlockSpec((HID, D), lambda i: (0, 0)),
        pl.BlockSpec((1, HID), lambda i: (0, 0)),
        pl.BlockSpec((NCLS, HID), lambda i: (0, 0)),
        pl.BlockSpec((1, NCLS), lambda i: (0, 0)),
    ],
    out_specs=pl.BlockSpec((MB, NCLS), lambda i: (i, 0)),
    out_shape=jax.ShapeDtypeStruct((B, NCLS), jnp.float32),
)


def kernel(x, emb_table, W1, b1, W2, b2):
    xr = x.astype(jnp.int32).reshape(NW, IDX_ROWS, GATHER_N)
    pooled_sum = _sc_pool(xr, emb_table)
    return _mlp(pooled_sum, W1, b1.reshape(1, HID), W2, b2.reshape(1, NCLS))


# speedup vs baseline: 1.0080x; 1.0080x over previous
import functools

import jax
import jax.numpy as jnp
from jax import lax
from jax.experimental import pallas as pl
from jax.experimental.pallas import tpu as pltpu
from jax.experimental.pallas import tpu_sc as plsc

VOCAB = 1000000
D = 64
NCLS = 16
B = 16384
SEQ = 20
HID = 32

NC = 2
NS = 16
NW = NC * NS

BPW = B // NW
CHUNK_B = 32
NCHUNK = BPW // CHUNK_B
CHUNK_ROWS = CHUNK_B * SEQ


def _sc_body(xt_hbm, table_hbm, out_hbm, idx_v, rows_v, out_v, sem0, sem1):
    cid = lax.axis_index("c")
    sid = lax.axis_index("s")
    wid = sid * NC + cid

    pltpu.sync_copy(xt_hbm.at[:, pl.ds(wid * BPW, BPW)], idx_v)

    def issue(c, par, sem):
        def g_body(j, carry):
            pltpu.async_copy(
                table_hbm.at[idx_v.at[j, pl.ds(c * CHUNK_B, CHUNK_B)]],
                rows_v.at[par, pl.ds(j * CHUNK_B, CHUNK_B)],
                sem,
            )
            return carry
        lax.fori_loop(0, SEQ, g_body, 0)

    def drain(par, sem):
        def w_body(j, carry):
            pltpu.make_async_copy(
                table_hbm.at[pl.ds(0, CHUNK_B), :],
                rows_v.at[par, pl.ds(0, CHUNK_B)],
                sem,
            ).wait()
            return carry
        lax.fori_loop(0, SEQ, w_body, 0)

    def pool(c, par):
        def b_body(b, carry):
            for dd in range(D // 16):
                acc = rows_v[par, b, pl.ds(dd * 16, 16)]
                for j in range(1, SEQ):
                    acc = acc + rows_v[par, j * CHUNK_B + b, pl.ds(dd * 16, 16)]
                out_v[c * CHUNK_B + b, pl.ds(dd * 16, 16)] = acc
            return carry
        lax.fori_loop(0, CHUNK_B, b_body, 0)

    issue(0, 0, sem0)

    def outer(k, carry):
        c0 = 2 * k
        c1 = 2 * k + 1
        issue(c1, 1, sem1)
        drain(0, sem0)
        pool(c0, 0)

        @pl.when(c1 + 1 < NCHUNK)
        def _():
            issue(c1 + 1, 0, sem0)

        drain(1, sem1)
        pool(c1, 1)
        return carry

    lax.fori_loop(0, NCHUNK // 2, outer, 0)

    pltpu.sync_copy(out_v, out_hbm.at[pl.ds(wid * BPW, BPW), :])


_sc_pool = functools.partial(
    pl.kernel,
    out_type=jax.ShapeDtypeStruct((B, D), jnp.float32),
    mesh=plsc.VectorSubcoreMesh(core_axis_name="c", subcore_axis_name="s"),
    compiler_params=pltpu.CompilerParams(use_tc_tiling_on_sc=False),
    scratch_types=[
        pltpu.VMEM((SEQ, BPW), jnp.int32),
        pltpu.VMEM((2, CHUNK_ROWS, D), jnp.float32),
        pltpu.VMEM((BPW, D), jnp.float32),
        pltpu.SemaphoreType.DMA,
        pltpu.SemaphoreType.DMA,
    ],
)(_sc_body)


MB = 2048


def _mlp_body(p_ref, w1_ref, b1_ref, w2_ref, b2_ref, o_ref):
    p = p_ref[...] * (1.0 / SEQ)
    h = lax.dot_general(p, w1_ref[...], (((1,), (1,)), ((), ())),
                        preferred_element_type=jnp.float32)
    h = h + b1_ref[...]
    o = lax.dot_general(h, w2_ref[...], (((1,), (1,)), ((), ())),
                        preferred_element_type=jnp.float32)
    o_ref[...] = o + b2_ref[...]


_mlp = pl.pallas_call(
    _mlp_body,
    grid=(B // MB,),
    in_specs=[
        pl.BlockSpec((MB, D), lambda i: (i, 0)),
        pl.BlockSpec((HID, D), lambda i: (0, 0)),
        pl.BlockSpec((1, HID), lambda i: (0, 0)),
        pl.BlockSpec((NCLS, HID), lambda i: (0, 0)),
        pl.BlockSpec((1, NCLS), lambda i: (0, 0)),
    ],
    out_specs=pl.BlockSpec((MB, NCLS), lambda i: (i, 0)),
    out_shape=jax.ShapeDtypeStruct((B, NCLS), jnp.float32),
)


def kernel(x, emb_table, W1, b1, W2, b2):
    xt = x.astype(jnp.int32).T
    pooled_sum = _sc_pool(xt, emb_table)
    return _mlp(pooled_sum, W1, b1.reshape(1, HID), W2, b2.reshape(1, NCLS))


# speedup vs baseline: 1.0853x; 1.0766x over previous
import functools

import jax
import jax.numpy as jnp
from jax import lax
from jax.experimental import pallas as pl
from jax.experimental.pallas import tpu as pltpu
from jax.experimental.pallas import tpu_sc as plsc

VOCAB = 1000000
D = 64
NCLS = 16
B = 16384
SEQ = 20
HID = 32

NC = 2
NS = 16
NW = NC * NS

BPW = B // NW
CHUNK_B = 16
NCHUNK = BPW // CHUNK_B
CHUNK_ROWS = CHUNK_B * SEQ
OPW = BPW // 2


def _sc_body(xt_hbm, table_hbm, out_hbm, idx_v, rows_v, out_v, sem0, sem1):
    cid = lax.axis_index("c")
    sid = lax.axis_index("s")
    wid = sid * NC + cid

    pltpu.sync_copy(xt_hbm.at[:, pl.ds(wid * BPW, BPW)], idx_v)

    def issue(c, par, sem):
        def g_body(j, carry):
            pltpu.async_copy(
                table_hbm.at[idx_v.at[j, pl.ds(c * CHUNK_B, CHUNK_B)]],
                rows_v.at[par, pl.ds(j * CHUNK_B, CHUNK_B)],
                sem,
            )
            return carry
        lax.fori_loop(0, SEQ, g_body, 0)

    def drain(par, sem):
        def w_body(j, carry):
            pltpu.make_async_copy(
                table_hbm.at[pl.ds(0, CHUNK_B), :],
                rows_v.at[par, pl.ds(0, CHUNK_B)],
                sem,
            ).wait()
            return carry
        lax.fori_loop(0, SEQ, w_body, 0)

    def pool(c, par):
        def b_body(bb, carry):
            for half in range(2):
                bag = 2 * bb + half
                for dd in range(D // 16):
                    acc = rows_v[par, bag, pl.ds(dd * 16, 16)]
                    for j in range(1, SEQ):
                        acc = acc + rows_v[par, j * CHUNK_B + bag,
                                           pl.ds(dd * 16, 16)]
                    out_v[c * (CHUNK_B // 2) + bb,
                          pl.ds(half * D + dd * 16, 16)] = acc
            return carry
        lax.fori_loop(0, CHUNK_B // 2, b_body, 0)

    issue(0, 0, sem0)

    def outer(k, carry):
        c0 = 2 * k
        c1 = 2 * k + 1
        issue(c1, 1, sem1)
        drain(0, sem0)
        pool(c0, 0)

        @pl.when(c1 + 1 < NCHUNK)
        def _():
            issue(c1 + 1, 0, sem0)

        drain(1, sem1)
        pool(c1, 1)
        return carry

    lax.fori_loop(0, NCHUNK // 2, outer, 0)

    pltpu.sync_copy(out_v, out_hbm.at[pl.ds(wid * OPW, OPW), :])


_sc_pool = functools.partial(
    pl.kernel,
    out_type=jax.ShapeDtypeStruct((B // 2, 2 * D), jnp.float32),
    mesh=plsc.VectorSubcoreMesh(core_axis_name="c", subcore_axis_name="s"),
    scratch_types=[
        pltpu.VMEM((SEQ, BPW), jnp.int32),
        pltpu.VMEM((2, CHUNK_ROWS, 2 * D), jnp.float32),
        pltpu.VMEM((OPW, 2 * D), jnp.float32),
        pltpu.SemaphoreType.DMA,
        pltpu.SemaphoreType.DMA,
    ],
)(_sc_body)


MB = 2048


def _mlp_body(p_ref, w1_ref, b1_ref, w2_ref, b2_ref, o_ref):
    p = p_ref[...] * (1.0 / SEQ)
    h = lax.dot_general(p, w1_ref[...], (((1,), (1,)), ((), ())),
                        preferred_element_type=jnp.float32)
    h = h + b1_ref[...]
    o = lax.dot_general(h, w2_ref[...], (((1,), (1,)), ((), ())),
                        preferred_element_type=jnp.float32)
    o_ref[...] = o + b2_ref[...]


_mlp = pl.pallas_call(
    _mlp_body,
    grid=(B // MB,),
    in_specs=[
        pl.BlockSpec((MB, D), lambda i: (i, 0)),
        pl.BlockSpec((HID, D), lambda i: (0, 0)),
        pl.BlockSpec((1, HID), lambda i: (0, 0)),
        pl.BlockSpec((NCLS, HID), lambda i: (0, 0)),
        pl.BlockSpec((1, NCLS), lambda i: (0, 0)),
    ],
    out_specs=pl.BlockSpec((MB, NCLS), lambda i: (i, 0)),
    out_shape=jax.ShapeDtypeStruct((B, NCLS), jnp.float32),
)


def kernel(x, emb_table, W1, b1, W2, b2):
    xt = x.astype(jnp.int32).T
    table_pad = jnp.pad(emb_table, ((0, 0), (0, D)))
    pooled_sum = _sc_pool(xt, table_pad).reshape(B, D)
    return _mlp(pooled_sum, W1, b1.reshape(1, HID), W2, b2.reshape(1, NCLS))
